# baseline (speedup 1.0000x reference)
import jax
import jax.numpy as jnp
from jax.experimental import pallas as pl
from jax.experimental.pallas import tpu as pltpu

BATCH = 16384
MAX_LEN = 150
NCH = 8
C = BATCH // NCH
NBUF = 4
PF = NBUF - 1


def _body(in_hbm, ids_hbm, mask_hbm, type_hbm,
          ibuf, mbuf, zbuf, in_sem, ids_sem, mask_sem, z_sem):
    def in_dma(i, s):
        return pltpu.make_async_copy(
            in_hbm.at[:, pl.ds(i * C, C)], ibuf.at[s], in_sem.at[s])

    def ids_dma(i, s):
        return pltpu.make_async_copy(
            ibuf.at[s], ids_hbm.at[:, pl.ds(i * C, C)], ids_sem.at[s])

    def mask_dma(i, s):
        return pltpu.make_async_copy(
            mbuf.at[s], mask_hbm.at[:, pl.ds(i * C, C)], mask_sem.at[s])

    def z_dma(i):
        return pltpu.make_async_copy(
            zbuf, type_hbm.at[:, pl.ds(i * C, C)], z_sem.at[i])

    for k in range(PF):
        in_dma(k, k).start()
    zbuf[...] = jnp.zeros_like(zbuf)
    for i in range(NCH):
        z_dma(i).start()
    for i in range(NCH):
        s = i % NBUF
        j = i + PF
        if j < NCH:
            sp = j % NBUF
            if i >= 1:
                ids_dma(i - 1, sp).wait()
            in_dma(j, sp).start()
        in_dma(i, s).wait()
        ids_dma(i, s).start()
        if i >= NBUF:
            mask_dma(i - NBUF, s).wait()
        mbuf[s] = jnp.where(ibuf[s] == 0, 0, 1).astype(jnp.int32)
        mask_dma(i, s).start()
    for i in range(NCH - PF - 1, NCH):
        ids_dma(i, i % NBUF).wait()
    for i in range(NCH - NBUF, NCH):
        mask_dma(i, i % NBUF).wait()
    for i in range(NCH):
        z_dma(i).wait()


def kernel(inputs):
    xt = inputs.T
    out_shape = jax.ShapeDtypeStruct((MAX_LEN, BATCH), jnp.int32)
    any_spec = pl.BlockSpec(memory_space=pl.ANY)
    ids, mask, type_ids = pl.pallas_call(
        _body,
        in_specs=[any_spec],
        out_specs=[any_spec, any_spec, any_spec],
        out_shape=[out_shape, out_shape, out_shape],
        scratch_shapes=[
            pltpu.VMEM((NBUF, MAX_LEN, C), jnp.int32),
            pltpu.VMEM((NBUF, MAX_LEN, C), jnp.int32),
            pltpu.VMEM((MAX_LEN, C), jnp.int32),
            pltpu.SemaphoreType.DMA((NBUF,)),
            pltpu.SemaphoreType.DMA((NBUF,)),
            pltpu.SemaphoreType.DMA((NBUF,)),
            pltpu.SemaphoreType.DMA((NCH,)),
        ],
    )(xt)
    return (ids.T, mask.T, type_ids.T)


# speedup vs baseline: 1.1027x; 1.1027x over previous
import jax
import jax.numpy as jnp
from jax.experimental import pallas as pl
from jax.experimental.pallas import tpu as pltpu

BATCH = 16384
MAX_LEN = 150
NCH = 4
C = BATCH // NCH


def _body(in_hbm, ids_hbm, mask_hbm, type_hbm,
          ibuf, mbuf, zbuf, in_sem, ids_sem, mask_sem, z_sem):
    def in_dma(i):
        return pltpu.make_async_copy(
            in_hbm.at[:, pl.ds(i * C, C)], ibuf.at[i], in_sem.at[i])

    def ids_dma(i):
        return pltpu.make_async_copy(
            ibuf.at[i], ids_hbm.at[:, pl.ds(i * C, C)], ids_sem.at[i])

    def mask_dma(i):
        return pltpu.make_async_copy(
            mbuf.at[i], mask_hbm.at[:, pl.ds(i * C, C)], mask_sem.at[i])

    def z_dma(i):
        return pltpu.make_async_copy(
            zbuf, type_hbm.at[:, pl.ds(i * C, C)], z_sem.at[i])

    for i in range(NCH):
        in_dma(i).start()
    zbuf[...] = jnp.zeros_like(zbuf)
    for i in range(NCH):
        z_dma(i).start()
    for i in range(NCH):
        in_dma(i).wait()
        ids_dma(i).start()
        mbuf[i] = jnp.where(ibuf[i] == 0, 0, 1).astype(jnp.int32)
        mask_dma(i).start()
    for i in range(NCH):
        ids_dma(i).wait()
        mask_dma(i).wait()
        z_dma(i).wait()


def kernel(inputs):
    xt = inputs.T
    out_shape = jax.ShapeDtypeStruct((MAX_LEN, BATCH), jnp.int32)
    any_spec = pl.BlockSpec(memory_space=pl.ANY)
    ids, mask, type_ids = pl.pallas_call(
        _body,
        in_specs=[any_spec],
        out_specs=[any_spec, any_spec, any_spec],
        out_shape=[out_shape, out_shape, out_shape],
        scratch_shapes=[
            pltpu.VMEM((NCH, MAX_LEN, C), jnp.int32),
            pltpu.VMEM((NCH, MAX_LEN, C), jnp.int32),
            pltpu.VMEM((MAX_LEN, C), jnp.int32),
            pltpu.SemaphoreType.DMA((NCH,)),
            pltpu.SemaphoreType.DMA((NCH,)),
            pltpu.SemaphoreType.DMA((NCH,)),
            pltpu.SemaphoreType.DMA((NCH,)),
        ],
    )(xt)
    return (ids.T, mask.T, type_ids.T)
